# Initial kernel scaffold; baseline (speedup 1.0000x reference)
import jax
import jax.numpy as jnp
from jax.experimental import pallas as pl


def kernel(x, ste, W_proj, b_proj, W_qk, W_v, W_out, b_out):
    raise NotImplementedError("write your pallas kernel here")


# speedup vs baseline: 5.1618x; 5.1618x over previous
import functools

import jax
import jax.numpy as jnp
from jax import lax
from jax.experimental import pallas as pl
from jax.experimental.pallas import tpu as pltpu
from jax.experimental.pallas import tpu_sc as plsc

BK = 16
NH = 4
HEADS = 8
T = 2080
NB = T // BK
TOT = NH * T
DH = 16
BH = 128
NW = 32
BH_W = BH // NW
NCH = NB * NH
G = 8
NG = NCH // G


def _qkv_body(x_ref, ste_ref, wp1_ref, wp2_ref, bp_ref, wqk_ref, wv_ref,
              rot_ref, qkh_ref, vh_ref, bkt_ref):
    xb = x_ref[0]
    sb = ste_ref[0]
    h = (jnp.dot(xb, wp1_ref[...], preferred_element_type=jnp.float32)
         + jnp.dot(sb, wp2_ref[...], preferred_element_type=jnp.float32)
         + bp_ref[...])
    qk = jnp.dot(h, wqk_ref[...], preferred_element_type=jnp.float32)
    v = jnp.dot(h, wv_ref[...], preferred_element_type=jnp.float32)
    lane = lax.broadcasted_iota(jnp.int32, (2048, 128), 1)
    in65 = lane < 65
    zpad = jnp.zeros((32, DH), jnp.float32)
    cols = []
    for j in range(HEADS):
        qh = qk[:, j * DH:(j + 1) * DH]
        qkh_ref[j, :2048, :] = qh
        qkh_ref[j, 2048:, :] = zpad
        vh_ref[j, :2048, :] = v[:, j * DH:(j + 1) * DH]
        vh_ref[j, 2048:, :] = zpad
        r = jnp.dot(qh, rot_ref[...], preferred_element_type=jnp.float32)
        for hh in range(NH):
            s = r[:, 128 * hh:128 * hh + 128]
            sm = jnp.where(in65, s, -1e30)
            sp = jnp.where(in65, s, 1e30)
            mx = jnp.max(sm, axis=1, keepdims=True)
            mn = jnp.min(sp, axis=1, keepdims=True)
            imax = jnp.min(jnp.where((s == mx) & in65, lane, 1000), axis=1)
            imin = jnp.min(jnp.where((s == mn) & in65, lane, 1000), axis=1)
            bkt = jnp.where(mx[:, 0] >= -mn[:, 0], imax, 65 + imin)
            cols.append(bkt[:, None].astype(jnp.int32))
    bkt_ref[0] = jnp.concatenate(cols, axis=1)


def _run_qkv(xr, ster, wp1, wp2, bp, wqkT, wvT, rotp):
    return pl.pallas_call(
        _qkv_body,
        grid=(16,),
        in_specs=[
            pl.BlockSpec((1, 2048, 128), lambda i: (i, 0, 0)),
            pl.BlockSpec((1, 2048, 64), lambda i: (i, 0, 0)),
            pl.BlockSpec((128, 128), lambda i: (0, 0)),
            pl.BlockSpec((64, 128), lambda i: (0, 0)),
            pl.BlockSpec((1, 128), lambda i: (0, 0)),
            pl.BlockSpec((128, 128), lambda i: (0, 0)),
            pl.BlockSpec((128, 128), lambda i: (0, 0)),
            pl.BlockSpec((16, 512), lambda i: (0, 0)),
        ],
        out_specs=[
            pl.BlockSpec((8, T, DH), lambda i: (i, 0, 0)),
            pl.BlockSpec((8, T, DH), lambda i: (i, 0, 0)),
            pl.BlockSpec((1, 2048, 32), lambda i: (i, 0, 0)),
        ],
        out_shape=[
            jax.ShapeDtypeStruct((BH, T, DH), jnp.float32),
            jax.ShapeDtypeStruct((BH, T, DH), jnp.float32),
            jax.ShapeDtypeStruct((16, 2048, 32), jnp.int32),
        ],
    )(xr, ster, wp1, wp2, bp, wqkT, wvT, rotp)


def _sort_body(bkt_hbm, qk_hbm, v_hbm, sqk_hbm, sv_hbm, st_hbm, rank_hbm,
               qk_v, v_v, bkt_v, occ_v, rank_v, st_v, idx_v, hist_v, cum_v,
               tmp_v, sem):
    wid = lax.axis_index("s") * 2 + lax.axis_index("c")
    lane = lax.iota(jnp.int32, 16)

    def do_hash(h, bhi):
        pltpu.sync_copy(bkt_hbm.at[bhi, h], bkt_v)
        for k in range(9):
            hist_v[pl.ds(k * 16, 16)] = jnp.zeros((16,), jnp.int32)

        def p1(c, carry):
            b16 = bkt_v[pl.ds(c * 16, 16)]
            key = b16 * 16 + lane
            sk, ls = plsc.sort_key_val(key, lane)
            tmp_v[...] = sk
            bs = sk >> 4
            prevk = plsc.load_gather(tmp_v, [jnp.maximum(lane - 1, 0)])
            new = (lane == 0) | ((prevk >> 4) != bs)
            runstart = plsc.cummax(jnp.where(new, lane, 0))
            occ_s = lane - runstart
            prior = plsc.load_gather(hist_v, [bs])
            plsc.addupdate_scatter(hist_v, [bs], jnp.ones((16,), jnp.int32))
            plsc.store_scatter(tmp_v, [ls], prior + occ_s)
            occ_v[pl.ds(c * 16, 16)] = tmp_v[...]
            return carry

        lax.fori_loop(0, NB, p1, 0)

        run = jnp.int32(0)
        for k in range(9):
            vv = hist_v[pl.ds(k * 16, 16)]
            cs = plsc.cumsum(vv)
            cum_v[pl.ds(k * 16, 16)] = cs - vv + run
            run = run + jnp.sum(vv)

        base = bhi * TOT + h * T

        def p3(c, carry):
            b16 = bkt_v[pl.ds(c * 16, 16)]
            occ16 = occ_v[pl.ds(c * 16, 16)]
            rk = plsc.load_gather(cum_v, [b16]) + occ16
            rank_v[pl.ds(c * 16, 16)] = rk
            plsc.store_scatter(st_v, [rk], lane + c * 16)
            row = c // 5
            cs0 = (c % 5) * 16
            idx_v[row, pl.ds(cs0, 16)] = rk + base
            return carry

        lax.fori_loop(0, NB, p3, 0)

        hs = []
        for r in range(26):
            hs.append(pltpu.async_copy(
                qk_v.at[pl.ds(r * 80, 80)], sqk_hbm.at[idx_v.at[r]], sem))
            hs.append(pltpu.async_copy(
                v_v.at[pl.ds(r * 80, 80)], sv_hbm.at[idx_v.at[r]], sem))
        for hnd in hs:
            hnd.wait()
        pltpu.sync_copy(st_v, st_hbm.at[bhi, h])
        pltpu.sync_copy(rank_v, rank_hbm.at[bhi, h])

    def do_bh(il, carry):
        bhi = wid * BH_W + il
        pltpu.sync_copy(qk_hbm.at[bhi], qk_v)
        pltpu.sync_copy(v_hbm.at[bhi], v_v)
        lax.fori_loop(0, NH, lambda h, c: (do_hash(h, bhi), c)[1], 0)
        return carry

    lax.fori_loop(0, BH_W, do_bh, 0)


def _run_sort(bkt_hl, qkh, vh):
    mesh = plsc.VectorSubcoreMesh(core_axis_name="c", subcore_axis_name="s")
    f = functools.partial(
        pl.kernel,
        out_type=(
            jax.ShapeDtypeStruct((BH * TOT, DH), jnp.float32),
            jax.ShapeDtypeStruct((BH * TOT, DH), jnp.float32),
            jax.ShapeDtypeStruct((BH, NH, T), jnp.int32),
            jax.ShapeDtypeStruct((BH, NH, T), jnp.int32),
        ),
        mesh=mesh,
        compiler_params=pltpu.CompilerParams(needs_layout_passes=False, use_tc_tiling_on_sc=False),
        scratch_types=[
            pltpu.VMEM((T, DH), jnp.float32),
            pltpu.VMEM((T, DH), jnp.float32),
            pltpu.VMEM((T,), jnp.int32),
            pltpu.VMEM((T,), jnp.int32),
            pltpu.VMEM((T,), jnp.int32),
            pltpu.VMEM((T,), jnp.int32),
            pltpu.VMEM((26, 80), jnp.int32),
            pltpu.VMEM((144,), jnp.int32),
            pltpu.VMEM((144,), jnp.int32),
            pltpu.VMEM((16,), jnp.int32),
            pltpu.SemaphoreType.DMA,
        ],
    )(_sort_body)
    return f(bkt_hl, qkh, vh)


def _attn_body(sqk_ref, sqkr_ref, svr_ref, stc_ref, strl_ref, so_ref):
    colid = lax.broadcasted_iota(jnp.int32, (128, 144), 1)
    rowid = lax.broadcasted_iota(jnp.int32, (128, 144), 0)
    diff = (colid >> 4) - (rowid >> 4)
    valid = ((diff >= 0) & (diff <= 1)).astype(jnp.float32)

    def body(g, carry):
        gs = g * 128
        Q = sqk_ref[0, pl.ds(gs, 128), :]
        K = sqkr_ref[0, pl.ds(gs, 144), :]
        V = svr_ref[0, pl.ds(gs, 144), :]
        n2 = jnp.sum(K * K, axis=1, keepdims=True)
        K = K * (1.0 / jnp.maximum(jnp.sqrt(n2), 1e-12))
        dots = lax.dot_general(Q, K, (((1,), (1,)), ((), ())),
                               preferred_element_type=jnp.float32) * 0.25
        qt = stc_ref[0, pl.ds(gs, 128), :]
        kt = strl_ref[0, 0, pl.ds(gs, 144)]
        dots = jnp.where(qt == kt[None, :], -5e4, dots)
        m = jnp.max(dots, axis=1, keepdims=True)
        e = jnp.exp(dots - m) * valid
        s = jnp.sum(e, axis=1, keepdims=True)
        lse = m + jnp.log(s)
        o = jnp.dot(e, V, preferred_element_type=jnp.float32) / s
        so_ref[0, pl.ds(gs, 128), 0:16] = o
        so_ref[0, pl.ds(gs, 128), 16:32] = jnp.broadcast_to(lse, (128, 16))
        return carry

    lax.fori_loop(0, NG, body, 0)


def _run_attn(sqk, sqk_r, sv_r, st_col, st_rl):
    return pl.pallas_call(
        _attn_body,
        grid=(BH,),
        in_specs=[
            pl.BlockSpec((1, TOT, DH), lambda i: (i, 0, 0)),
            pl.BlockSpec((1, TOT + BK, DH), lambda i: (i, 0, 0)),
            pl.BlockSpec((1, TOT + BK, DH), lambda i: (i, 0, 0)),
            pl.BlockSpec((1, TOT, 1), lambda i: (i, 0, 0)),
            pl.BlockSpec((1, 1, TOT + BK), lambda i: (i, 0, 0)),
        ],
        out_specs=pl.BlockSpec((1, TOT, 32), lambda i: (i, 0, 0)),
        out_shape=jax.ShapeDtypeStruct((BH, TOT, 32), jnp.float32),
    )(sqk, sqk_r, sv_r, st_col, st_rl)


def _unsort_body(so_hbm, rank_hbm, out_hbm, rank4_v, buf_v, idx_v, w_v, tmp_v,
                 out_v, sem):
    wid = lax.axis_index("s") * 2 + lax.axis_index("c")
    lane = lax.iota(jnp.int32, 16)
    col16 = jnp.full((16,), 16, jnp.int32)

    def do_bh(il, carry):
        bhi = wid * BH_W + il
        pltpu.sync_copy(rank_hbm.at[bhi], rank4_v)
        base = bhi * TOT

        def blk(bk, c2):
            t0 = bk * 20
            for sub in range(5):
                tl = t0 + sub * 4 + lane // 4
                hl = lane % 4
                rv = plsc.load_gather(rank4_v, [hl, tl])
                idx_v[pl.ds(sub * 16, 16)] = base + hl * T + rv
            pltpu.async_copy(so_hbm.at[idx_v], buf_v, sem).wait()
            for g in range(5):
                rows = g * 16 + lane
                L0 = plsc.load_gather(buf_v, [rows, col16])
                L1 = plsc.load_gather(buf_v, [g * 16 + (lane ^ 1), col16])
                L2 = plsc.load_gather(buf_v, [g * 16 + (lane ^ 2), col16])
                L3 = plsc.load_gather(buf_v, [g * 16 + (lane ^ 3), col16])
                m = jnp.maximum(jnp.maximum(L0, L1), jnp.maximum(L2, L3))
                w = jnp.exp(L0 - m)
                s = w + jnp.exp(L1 - m) + jnp.exp(L2 - m) + jnp.exp(L3 - m)
                rw = w / s
                for tt in range(4):
                    rbase = g * 16 + tt * 4
                    acc = None
                    for hh in range(4):
                        ws = jnp.sum(jnp.where(lane == tt * 4 + hh, rw, 0.0))
                        term = ws * buf_v[rbase + hh, 0:16]
                        acc = term if acc is None else acc + term
                    out_v[t0 + g * 4 + tt, :] = acc
            return c2

        lax.fori_loop(0, 104, blk, 0)
        pltpu.sync_copy(out_v, out_hbm.at[bhi])
        return carry

    lax.fori_loop(0, BH_W, do_bh, 0)


def _run_unsort(so_flat, rank):
    mesh = plsc.VectorSubcoreMesh(core_axis_name="c", subcore_axis_name="s")
    f = functools.partial(
        pl.kernel,
        out_type=jax.ShapeDtypeStruct((BH, T, DH), jnp.float32),
        mesh=mesh,
        compiler_params=pltpu.CompilerParams(needs_layout_passes=False, use_tc_tiling_on_sc=False),
        scratch_types=[
            pltpu.VMEM((NH, T), jnp.int32),
            pltpu.VMEM((80, 32), jnp.float32),
            pltpu.VMEM((80,), jnp.int32),
            pltpu.VMEM((16,), jnp.float32),
            pltpu.VMEM((16,), jnp.float32),
            pltpu.VMEM((T, DH), jnp.float32),
            pltpu.SemaphoreType.DMA,
        ],
    )(_unsort_body)
    return f(so_flat, rank)


def _out_body(d_ref, wout_ref, bout_ref, out_ref):
    hcat = jnp.concatenate([d_ref[j, :2048, :] for j in range(HEADS)], axis=1)
    out_ref[0] = (jnp.dot(hcat, wout_ref[...], preferred_element_type=jnp.float32)
                  + bout_ref[...])


def _run_out(out_h, woutT, bout):
    return pl.pallas_call(
        _out_body,
        grid=(16,),
        in_specs=[
            pl.BlockSpec((8, T, DH), lambda i: (i, 0, 0)),
            pl.BlockSpec((128, 128), lambda i: (0, 0)),
            pl.BlockSpec((1, 128), lambda i: (0, 0)),
        ],
        out_specs=pl.BlockSpec((1, 2048, 128), lambda i: (i, 0, 0)),
        out_shape=jax.ShapeDtypeStruct((16, 2048, 128), jnp.float32),
    )(out_h, woutT, bout)


def kernel(x, ste, W_proj, b_proj, W_qk, W_v, W_out, b_out):
    b, l, n, df = x.shape
    xr = x.reshape(16, 2048, 128)
    ster = ste.reshape(16, 2048, 64)
    wp1 = W_proj[:, :128].T
    wp2 = W_proj[:, 128:].T
    bp = b_proj.reshape(1, 128)
    rot = jax.random.normal(jax.random.key(42), (DH, NH, NB // 2),
                            dtype=jnp.float32)
    rotp = jnp.zeros((DH, NH, 128), jnp.float32).at[:, :, :65].set(rot)
    rotp = rotp.reshape(DH, NH * 128)

    qkh, vh, bkt = _run_qkv(xr, ster, wp1, wp2, bp, W_qk.T, W_v.T, rotp)

    bkt_hl = bkt.reshape(16, 2048, 8, 4).transpose(0, 2, 3, 1).reshape(BH, NH, 2048)
    bkt_hl = jnp.pad(bkt_hl, ((0, 0), (0, 0), (0, 32)))

    sqk_f, sv_f, st, rank = _run_sort(bkt_hl, qkh, vh)
    sqk = sqk_f.reshape(BH, TOT, DH)
    sv = sv_f.reshape(BH, TOT, DH)
    stf = st.reshape(BH, TOT)

    sqk_r = jnp.concatenate([sqk[:, TOT - BK:], sqk], axis=1)
    sv_r = jnp.concatenate([sv[:, TOT - BK:], sv], axis=1)
    st_r = jnp.concatenate([stf[:, TOT - BK:], stf], axis=1)

    so = _run_attn(sqk, sqk_r, sv_r, stf.reshape(BH, TOT, 1),
                   st_r.reshape(BH, 1, TOT + BK))

    out_h = _run_unsort(so.reshape(BH * TOT, 32), rank)

    out = _run_out(out_h, W_out.T, b_out.reshape(1, 128))
    return out.reshape(b, l, n, 128)


# speedup vs baseline: 5.6291x; 1.0905x over previous
import functools

import jax
import jax.numpy as jnp
from jax import lax
from jax.experimental import pallas as pl
from jax.experimental.pallas import tpu as pltpu
from jax.experimental.pallas import tpu_sc as plsc

BK = 16
NH = 4
HEADS = 8
T = 2080
NB = T // BK
TOT = NH * T
DH = 16
BH = 128
NW = 32
BH_W = BH // NW
NCH = NB * NH
G = 8
NG = NCH // G


def _qkv_body(x_ref, ste_ref, wp1_ref, wp2_ref, bp_ref, wqk_ref, wv_ref,
              rot_ref, qkh_ref, vh_ref, bkt_ref):
    xb = x_ref[0]
    sb = ste_ref[0]
    h = (jnp.dot(xb, wp1_ref[...], preferred_element_type=jnp.float32)
         + jnp.dot(sb, wp2_ref[...], preferred_element_type=jnp.float32)
         + bp_ref[...])
    qk = jnp.dot(h, wqk_ref[...], preferred_element_type=jnp.float32)
    v = jnp.dot(h, wv_ref[...], preferred_element_type=jnp.float32)
    lane = lax.broadcasted_iota(jnp.int32, (2048, 128), 1)
    in65 = lane < 65
    zpad = jnp.zeros((32, DH), jnp.float32)
    cols = []
    for j in range(HEADS):
        qh = qk[:, j * DH:(j + 1) * DH]
        qkh_ref[j, :2048, :] = qh
        qkh_ref[j, 2048:, :] = zpad
        vh_ref[j, :2048, :] = v[:, j * DH:(j + 1) * DH]
        vh_ref[j, 2048:, :] = zpad
        r = jnp.dot(qh, rot_ref[...], preferred_element_type=jnp.float32)
        for hh in range(NH):
            s = r[:, 128 * hh:128 * hh + 128]
            sm = jnp.where(in65, s, -1e30)
            sp = jnp.where(in65, s, 1e30)
            mx = jnp.max(sm, axis=1, keepdims=True)
            mn = jnp.min(sp, axis=1, keepdims=True)
            imax = jnp.min(jnp.where((s == mx) & in65, lane, 1000), axis=1)
            imin = jnp.min(jnp.where((s == mn) & in65, lane, 1000), axis=1)
            bkt = jnp.where(mx[:, 0] >= -mn[:, 0], imax, 65 + imin)
            cols.append(bkt[:, None].astype(jnp.int32))
    bkt_ref[0] = jnp.concatenate(cols, axis=1)


def _run_qkv(xr, ster, wp1, wp2, bp, wqkT, wvT, rotp):
    return pl.pallas_call(
        _qkv_body,
        grid=(16,),
        in_specs=[
            pl.BlockSpec((1, 2048, 128), lambda i: (i, 0, 0)),
            pl.BlockSpec((1, 2048, 64), lambda i: (i, 0, 0)),
            pl.BlockSpec((128, 128), lambda i: (0, 0)),
            pl.BlockSpec((64, 128), lambda i: (0, 0)),
            pl.BlockSpec((1, 128), lambda i: (0, 0)),
            pl.BlockSpec((128, 128), lambda i: (0, 0)),
            pl.BlockSpec((128, 128), lambda i: (0, 0)),
            pl.BlockSpec((16, 512), lambda i: (0, 0)),
        ],
        out_specs=[
            pl.BlockSpec((8, T, DH), lambda i: (i, 0, 0)),
            pl.BlockSpec((8, T, DH), lambda i: (i, 0, 0)),
            pl.BlockSpec((1, 2048, 32), lambda i: (i, 0, 0)),
        ],
        out_shape=[
            jax.ShapeDtypeStruct((BH, T, DH), jnp.float32),
            jax.ShapeDtypeStruct((BH, T, DH), jnp.float32),
            jax.ShapeDtypeStruct((16, 2048, 32), jnp.int32),
        ],
    )(xr, ster, wp1, wp2, bp, wqkT, wvT, rotp)


def _sort_body(bkt_hbm, qk_hbm, v_hbm, sqk_hbm, sv_hbm, st_hbm, rank_hbm,
               qk_v, v_v, bkt_v, occ_v, rank_v, st_v, idx_v, hist_v, cum_v,
               tmp_v, sem):
    wid = lax.axis_index("s") * 2 + lax.axis_index("c")
    lane = lax.iota(jnp.int32, 16)

    def do_hash(h, bhi):
        pltpu.sync_copy(bkt_hbm.at[bhi, h], bkt_v)
        for k in range(9):
            hist_v[pl.ds(k * 16, 16)] = jnp.zeros((16,), jnp.int32)

        def p1(c, carry):
            b16 = bkt_v[pl.ds(c * 16, 16)]
            key = b16 * 16 + lane
            sk, ls = plsc.sort_key_val(key, lane)
            tmp_v[...] = sk
            bs = sk >> 4
            prevk = plsc.load_gather(tmp_v, [jnp.maximum(lane - 1, 0)])
            new = (lane == 0) | ((prevk >> 4) != bs)
            runstart = plsc.cummax(jnp.where(new, lane, 0))
            occ_s = lane - runstart
            prior = plsc.load_gather(hist_v, [bs])
            plsc.addupdate_scatter(hist_v, [bs], jnp.ones((16,), jnp.int32))
            plsc.store_scatter(tmp_v, [ls], prior + occ_s)
            occ_v[pl.ds(c * 16, 16)] = tmp_v[...]
            return carry

        lax.fori_loop(0, NB, p1, 0)

        run = jnp.int32(0)
        for k in range(9):
            vv = hist_v[pl.ds(k * 16, 16)]
            cs = plsc.cumsum(vv)
            cum_v[pl.ds(k * 16, 16)] = cs - vv + run
            run = run + jnp.sum(vv)

        base = bhi * TOT + h * T

        def p3(c, carry):
            b16 = bkt_v[pl.ds(c * 16, 16)]
            occ16 = occ_v[pl.ds(c * 16, 16)]
            rk = plsc.load_gather(cum_v, [b16]) + occ16
            rank_v[pl.ds(c * 16, 16)] = rk
            plsc.store_scatter(st_v, [rk], lane + c * 16)
            row = c // 5
            cs0 = (c % 5) * 16
            idx_v[row, pl.ds(cs0, 16)] = rk + base
            return carry

        lax.fori_loop(0, NB, p3, 0)

        hs = []
        for r in range(26):
            hs.append(pltpu.async_copy(
                qk_v.at[pl.ds(r * 80, 80)], sqk_hbm.at[idx_v.at[r]], sem))
            hs.append(pltpu.async_copy(
                v_v.at[pl.ds(r * 80, 80)], sv_hbm.at[idx_v.at[r]], sem))
        for hnd in hs:
            hnd.wait()
        pltpu.sync_copy(st_v, st_hbm.at[bhi, h])
        pltpu.sync_copy(rank_v, rank_hbm.at[bhi, h])

    def do_bh(il, carry):
        bhi = wid * BH_W + il
        pltpu.sync_copy(qk_hbm.at[bhi], qk_v)
        pltpu.sync_copy(v_hbm.at[bhi], v_v)
        lax.fori_loop(0, NH, lambda h, c: (do_hash(h, bhi), c)[1], 0)
        return carry

    lax.fori_loop(0, BH_W, do_bh, 0)


def _run_sort(bkt_hl, qkh, vh):
    mesh = plsc.VectorSubcoreMesh(core_axis_name="c", subcore_axis_name="s")
    f = functools.partial(
        pl.kernel,
        out_type=(
            jax.ShapeDtypeStruct((BH * TOT, DH), jnp.float32),
            jax.ShapeDtypeStruct((BH * TOT, DH), jnp.float32),
            jax.ShapeDtypeStruct((BH, NH, T), jnp.int32),
            jax.ShapeDtypeStruct((BH, NH, T), jnp.int32),
        ),
        mesh=mesh,
        compiler_params=pltpu.CompilerParams(needs_layout_passes=False, use_tc_tiling_on_sc=False),
        scratch_types=[
            pltpu.VMEM((T, DH), jnp.float32),
            pltpu.VMEM((T, DH), jnp.float32),
            pltpu.VMEM((T,), jnp.int32),
            pltpu.VMEM((T,), jnp.int32),
            pltpu.VMEM((T,), jnp.int32),
            pltpu.VMEM((T,), jnp.int32),
            pltpu.VMEM((26, 80), jnp.int32),
            pltpu.VMEM((144,), jnp.int32),
            pltpu.VMEM((144,), jnp.int32),
            pltpu.VMEM((16,), jnp.int32),
            pltpu.SemaphoreType.DMA,
        ],
    )(_sort_body)
    return f(bkt_hl, qkh, vh)


def _attn_body(sqk_ref, sv_ref, stc_ref, strl_ref, so_ref):
    colid = lax.broadcasted_iota(jnp.int32, (128, 144), 1)
    rowid = lax.broadcasted_iota(jnp.int32, (128, 144), 0)
    diff = (colid >> 4) - (rowid >> 4)
    valid = ((diff >= 0) & (diff <= 1)).astype(jnp.float32)

    def body(g, carry):
        gs = g * 128
        ps = jnp.where(g == 0, TOT - BK, gs - BK)
        Q = sqk_ref[0, pl.ds(gs, 128), :]
        K = jnp.concatenate([sqk_ref[0, pl.ds(ps, BK), :], Q], axis=0)
        V = jnp.concatenate([sv_ref[0, pl.ds(ps, BK), :],
                             sv_ref[0, pl.ds(gs, 128), :]], axis=0)
        n2 = jnp.sum(K * K, axis=1, keepdims=True)
        K = K * (1.0 / jnp.maximum(jnp.sqrt(n2), 1e-12))
        dots = lax.dot_general(Q, K, (((1,), (1,)), ((), ())),
                               preferred_element_type=jnp.float32) * 0.25
        qt = stc_ref[0, pl.ds(gs, 128), :]
        kt = strl_ref[0, 0, pl.ds(gs, 144)]
        dots = jnp.where(qt == kt[None, :], -5e4, dots)
        m = jnp.max(dots, axis=1, keepdims=True)
        e = jnp.exp(dots - m) * valid
        s = jnp.sum(e, axis=1, keepdims=True)
        lse = m + jnp.log(s)
        o = jnp.dot(e, V, preferred_element_type=jnp.float32) / s
        so_ref[0, pl.ds(gs, 128), 0:16] = o
        so_ref[0, pl.ds(gs, 128), 16:32] = jnp.broadcast_to(lse, (128, 16))
        return carry

    lax.fori_loop(0, NG, body, 0)


def _run_attn(sqk, sv, st_col, st_rl):
    return pl.pallas_call(
        _attn_body,
        grid=(BH,),
        in_specs=[
            pl.BlockSpec((1, TOT, DH), lambda i: (i, 0, 0)),
            pl.BlockSpec((1, TOT, DH), lambda i: (i, 0, 0)),
            pl.BlockSpec((1, TOT, 1), lambda i: (i, 0, 0)),
            pl.BlockSpec((1, 1, TOT + BK), lambda i: (i, 0, 0)),
        ],
        out_specs=pl.BlockSpec((1, TOT, 32), lambda i: (i, 0, 0)),
        out_shape=jax.ShapeDtypeStruct((BH, TOT, 32), jnp.float32),
    )(sqk, sv, st_col, st_rl)


def _unsort_body(so_hbm, rank_hbm, out_hbm, rank4_v, buf_v, idx_v, w_v, tmp_v,
                 out_v, sem):
    wid = lax.axis_index("s") * 2 + lax.axis_index("c")
    lane = lax.iota(jnp.int32, 16)
    col16 = jnp.full((16,), 16, jnp.int32)

    def do_bh(il, carry):
        bhi = wid * BH_W + il
        pltpu.sync_copy(rank_hbm.at[bhi], rank4_v)
        base = bhi * TOT

        def blk(bk, c2):
            t0 = bk * 20
            for sub in range(5):
                tl = t0 + sub * 4 + lane // 4
                hl = lane % 4
                rv = plsc.load_gather(rank4_v, [hl, tl])
                idx_v[pl.ds(sub * 16, 16)] = base + hl * T + rv
            pltpu.async_copy(so_hbm.at[idx_v], buf_v, sem).wait()
            for g in range(5):
                rows = g * 16 + lane
                L0 = plsc.load_gather(buf_v, [rows, col16])
                L1 = plsc.load_gather(buf_v, [g * 16 + (lane ^ 1), col16])
                L2 = plsc.load_gather(buf_v, [g * 16 + (lane ^ 2), col16])
                L3 = plsc.load_gather(buf_v, [g * 16 + (lane ^ 3), col16])
                m = jnp.maximum(jnp.maximum(L0, L1), jnp.maximum(L2, L3))
                w = jnp.exp(L0 - m)
                s = w + jnp.exp(L1 - m) + jnp.exp(L2 - m) + jnp.exp(L3 - m)
                rw = w / s
                for tt in range(4):
                    rbase = g * 16 + tt * 4
                    acc = None
                    for hh in range(4):
                        ws = jnp.sum(jnp.where(lane == tt * 4 + hh, rw, 0.0))
                        term = ws * buf_v[rbase + hh, 0:16]
                        acc = term if acc is None else acc + term
                    out_v[t0 + g * 4 + tt, :] = acc
            return c2

        lax.fori_loop(0, 104, blk, 0)
        pltpu.sync_copy(out_v, out_hbm.at[bhi])
        return carry

    lax.fori_loop(0, BH_W, do_bh, 0)


def _run_unsort(so_flat, rank):
    mesh = plsc.VectorSubcoreMesh(core_axis_name="c", subcore_axis_name="s")
    f = functools.partial(
        pl.kernel,
        out_type=jax.ShapeDtypeStruct((BH, T, DH), jnp.float32),
        mesh=mesh,
        compiler_params=pltpu.CompilerParams(needs_layout_passes=False, use_tc_tiling_on_sc=False),
        scratch_types=[
            pltpu.VMEM((NH, T), jnp.int32),
            pltpu.VMEM((80, 32), jnp.float32),
            pltpu.VMEM((80,), jnp.int32),
            pltpu.VMEM((16,), jnp.float32),
            pltpu.VMEM((16,), jnp.float32),
            pltpu.VMEM((T, DH), jnp.float32),
            pltpu.SemaphoreType.DMA,
        ],
    )(_unsort_body)
    return f(so_flat, rank)


def _out_body(d_ref, wout_ref, bout_ref, out_ref):
    hcat = jnp.concatenate([d_ref[j, :2048, :] for j in range(HEADS)], axis=1)
    out_ref[0] = (jnp.dot(hcat, wout_ref[...], preferred_element_type=jnp.float32)
                  + bout_ref[...])


def _run_out(out_h, woutT, bout):
    return pl.pallas_call(
        _out_body,
        grid=(16,),
        in_specs=[
            pl.BlockSpec((8, T, DH), lambda i: (i, 0, 0)),
            pl.BlockSpec((128, 128), lambda i: (0, 0)),
            pl.BlockSpec((1, 128), lambda i: (0, 0)),
        ],
        out_specs=pl.BlockSpec((1, 2048, 128), lambda i: (i, 0, 0)),
        out_shape=jax.ShapeDtypeStruct((16, 2048, 128), jnp.float32),
    )(out_h, woutT, bout)


def kernel(x, ste, W_proj, b_proj, W_qk, W_v, W_out, b_out):
    b, l, n, df = x.shape
    xr = x.reshape(16, 2048, 128)
    ster = ste.reshape(16, 2048, 64)
    wp1 = W_proj[:, :128].T
    wp2 = W_proj[:, 128:].T
    bp = b_proj.reshape(1, 128)
    rot = jax.random.normal(jax.random.key(42), (DH, NH, NB // 2),
                            dtype=jnp.float32)
    rotp = jnp.zeros((DH, NH, 128), jnp.float32).at[:, :, :65].set(rot)
    rotp = rotp.reshape(DH, NH * 128)

    qkh, vh, bkt = _run_qkv(xr, ster, wp1, wp2, bp, W_qk.T, W_v.T, rotp)

    bkt_hl = bkt.reshape(16, 2048, 8, 4).transpose(0, 2, 3, 1).reshape(BH, NH, 2048)
    bkt_hl = jnp.pad(bkt_hl, ((0, 0), (0, 0), (0, 32)))

    sqk_f, sv_f, st, rank = _run_sort(bkt_hl, qkh, vh)
    sqk = sqk_f.reshape(BH, TOT, DH)
    sv = sv_f.reshape(BH, TOT, DH)
    stf = st.reshape(BH, TOT)

    st_r = jnp.concatenate([stf[:, TOT - BK:], stf], axis=1)
    so = _run_attn(sqk, sv, stf.reshape(BH, TOT, 1),
                   st_r.reshape(BH, 1, TOT + BK))

    out_h = _run_unsort(so.reshape(BH * TOT, 32), rank)

    out = _run_out(out_h, W_out.T, b_out.reshape(1, 128))
    return out.reshape(b, l, n, 128)
